# baseline (speedup 1.0000x reference)
import functools

import jax
import jax.numpy as jnp
from jax import lax
from jax.experimental import pallas as pl
from jax.experimental.pallas import tpu as pltpu
from jax.experimental.pallas import tpu_sc as plsc

V = 10000
VP = 10240
L = 16
NW = 32
GK = 128
NB = 4


def _make_sc_gather(CW: int):
    R = VP * L
    chunk = R // NW
    iters = chunk // GK
    rounds = iters // NB
    rows_per_tile = VP // 16
    mesh = plsc.VectorSubcoreMesh(core_axis_name="c", subcore_axis_name="s")

    @functools.partial(
        pl.kernel,
        mesh=mesh,
        out_type=jax.ShapeDtypeStruct((R, CW), jnp.int32),
        scratch_types=[
            pltpu.VMEM((chunk,), jnp.int32),
        ] + [pltpu.VMEM((GK, CW), jnp.int32) for _ in range(NB)]
          + [pltpu.SemaphoreType.DMA for _ in range(2 * NB)],
    )
    def kfn(table_hbm, idx_hbm, g_hbm, idx_v, *rest):
        bufs = rest[:NB]
        gsems = rest[NB:2 * NB]
        wsems = rest[2 * NB:]
        cid = lax.axis_index("c")
        sid = lax.axis_index("s")
        wid = sid * 2 + cid
        base = pl.multiple_of(wid * chunk, GK)
        pltpu.sync_copy(idx_hbm.at[pl.ds(base, chunk)], idx_v)

        def start_gather(b, g):
            pltpu.async_copy(
                table_hbm.at[idx_v.at[pl.ds(g * GK, GK)]], bufs[b], gsems[b])

        def wait_gather(b, g):
            del g
            pltpu.make_async_copy(
                g_hbm.at[pl.ds(0, GK)], bufs[b], gsems[b]).wait()

        def start_wb(b, g):
            off = pl.multiple_of(base + g * GK, GK)
            pltpu.async_copy(bufs[b], g_hbm.at[pl.ds(off, GK)], wsems[b])

        def wait_wb(b):
            pltpu.make_async_copy(
                bufs[b], g_hbm.at[pl.ds(0, GK)], wsems[b]).wait()

        for b in range(NB):
            start_gather(b, b)

        def body(r, carry):
            for b in range(NB):
                g = r * NB + b
                wait_gather(b, g)
                start_wb(b, g)
            for b in range(NB):
                wait_wb(b)
                start_gather(b, (r + 1) * NB + b)
            return carry

        lax.fori_loop(0, rounds - 1, body, 0)
        for b in range(NB):
            g = (rounds - 1) * NB + b
            wait_gather(b, g)
            start_wb(b, g)
        for b in range(NB):
            wait_wb(b)

    return kfn


def _to_words(a):
    M, C = a.shape
    return lax.bitcast_convert_type(a.reshape(M, C // 2, 2), jnp.int32)


def _from_words(w, C):
    M = w.shape[0]
    return lax.bitcast_convert_type(w, jnp.bfloat16).reshape(M, C)


def _mm(xp, W, bias8=None, relu=False, res=None, mask=True,
        out_dtype=jnp.bfloat16, BM=256):
    M, K = xp.shape
    N = W.shape[1]
    nblk = M // BM
    args = [xp, W]
    in_specs = [
        pl.BlockSpec((BM, K), lambda i: (i, 0)),
        pl.BlockSpec((K, N), lambda i: (0, 0)),
    ]
    if bias8 is not None:
        args.append(bias8)
        in_specs.append(pl.BlockSpec((8, N), lambda i: (0, 0)))
    if res is not None:
        args.append(res)
        in_specs.append(pl.BlockSpec((BM, N), lambda i: (i, 0)))
    have_bias = bias8 is not None
    have_res = res is not None

    def body(*refs):
        x_ref, w_ref = refs[0], refs[1]
        rest = refs[2:-1]
        o_ref = refs[-1]
        y = jnp.dot(x_ref[...].astype(jnp.bfloat16), w_ref[...],
                    preferred_element_type=jnp.float32)
        ri = 0
        if have_bias:
            y = y + rest[0][0:1, :]
            ri = 1
        if mask:
            i = pl.program_id(0)
            rows = i * BM + lax.broadcasted_iota(jnp.int32, (BM, 1), 0)
            y = jnp.where(rows < V, y, 0.0)
        if have_res:
            y = y + rest[ri][...].astype(jnp.float32)
        if relu:
            y = jnp.maximum(y, 0.0)
        o_ref[...] = y.astype(o_ref.dtype)

    return pl.pallas_call(
        body,
        grid=(nblk,),
        in_specs=in_specs,
        out_specs=pl.BlockSpec((BM, N), lambda i: (i, 0)),
        out_shape=jax.ShapeDtypeStruct((M, N), out_dtype),
    )(*args)


def _wd_and_max(fs, Wd, BM=256):
    M, K = fs.shape
    N = Wd.shape[1]
    nblk = M // BM

    def body(x_ref, w_ref, o_ref, m_ref):
        i = pl.program_id(0)
        y = jnp.dot(x_ref[...], w_ref[...], preferred_element_type=jnp.float32)
        rows = i * BM + lax.broadcasted_iota(jnp.int32, (BM, 1), 0)
        valid = rows < V
        yv = jnp.where(valid, y, 0.0)
        o_ref[...] = yv
        ym = jnp.where(valid, y, -1e30)
        m = ym[0:8]
        for j in range(1, BM // 8):
            m = jnp.maximum(m, ym[j * 8:(j + 1) * 8])

        @pl.when(i == 0)
        def _():
            m_ref[...] = m

        @pl.when(i > 0)
        def _():
            m_ref[...] = jnp.maximum(m_ref[...], m)

    return pl.pallas_call(
        body,
        grid=(nblk,),
        in_specs=[
            pl.BlockSpec((BM, K), lambda i: (i, 0)),
            pl.BlockSpec((K, N), lambda i: (0, 0)),
        ],
        out_specs=[
            pl.BlockSpec((BM, N), lambda i: (i, 0)),
            pl.BlockSpec((8, N), lambda i: (0, 0)),
        ],
        out_shape=[
            jax.ShapeDtypeStruct((M, N), jnp.float32),
            jax.ShapeDtypeStruct((8, N), jnp.float32),
        ],
    )(fs, Wd)


def _final_mlp(pfs, fs, macc, W1p, W1f, W1g, b1_8, W2, b2_8, W3p, b3_8,
               BM=256):
    M = pfs.shape[0]
    nblk = M // BM

    def body(p_ref, f_ref, g_ref, w1p, w1f, w1g, b1, w2, b2, w3, b3, o_ref):
        gmax = jnp.max(g_ref[...], axis=0, keepdims=True)
        gc = jnp.dot(gmax.astype(jnp.bfloat16), w1g[...],
                     preferred_element_type=jnp.float32)
        o1 = jnp.dot(p_ref[...], w1p[...], preferred_element_type=jnp.float32)
        o1 = o1 + jnp.dot(f_ref[...].astype(jnp.bfloat16), w1f[...],
                          preferred_element_type=jnp.float32)
        o1 = jnp.maximum(o1 + gc + b1[0:1, :], 0.0).astype(jnp.bfloat16)
        o2 = jnp.maximum(
            jnp.dot(o1, w2[...], preferred_element_type=jnp.float32)
            + b2[0:1, :], 0.0).astype(jnp.bfloat16)
        o_ref[...] = (
            jnp.dot(o2, w3[...], preferred_element_type=jnp.float32)
            + b3[0:1, :])

    specs = [
        pl.BlockSpec((BM, 256), lambda i: (i, 0)),
        pl.BlockSpec((BM, 128), lambda i: (i, 0)),
        pl.BlockSpec((8, 128), lambda i: (0, 0)),
        pl.BlockSpec((256, 256), lambda i: (0, 0)),
        pl.BlockSpec((128, 256), lambda i: (0, 0)),
        pl.BlockSpec((128, 256), lambda i: (0, 0)),
        pl.BlockSpec((8, 256), lambda i: (0, 0)),
        pl.BlockSpec((256, 128), lambda i: (0, 0)),
        pl.BlockSpec((8, 128), lambda i: (0, 0)),
        pl.BlockSpec((128, 128), lambda i: (0, 0)),
        pl.BlockSpec((8, 128), lambda i: (0, 0)),
    ]
    return pl.pallas_call(
        body,
        grid=(nblk,),
        in_specs=specs,
        out_specs=pl.BlockSpec((BM, 128), lambda i: (i, 0)),
        out_shape=jax.ShapeDtypeStruct((M, 128), jnp.float32),
    )(pfs, fs, macc, W1p, W1f, W1g, b1_8, W2, b2_8, W3p, b3_8)


def _b8(b):
    return jnp.broadcast_to(b.reshape(1, -1), (8, b.shape[0]))


def _bf(a):
    return a.astype(jnp.bfloat16)


def kernel(x, spiral, Wp, W1a, b1a, W1b, b1b, Wd, Wr0a, br0a, Wr0b, br0b,
           Wr1a, br1a, Wr1b, br1b, Wr2a, br2a, Wr2b, br2b, Wo1, bo1, Wo2,
           bo2, Wo3, bo3):
    Bn, Vn, FIN = x.shape
    KP = 512
    xp = _bf(jnp.pad(x[0], ((0, VP - Vn), (0, KP - FIN))))
    Wpp = _bf(jnp.pad(Wp, ((0, KP - FIN), (0, 0))))
    idxf = jnp.pad(spiral.reshape(-1), (0, VP * L - spiral.size))
    idxf = idxf.astype(jnp.int32)

    gather512b = _make_sc_gather(128)

    pfs = _mm(xp, Wpp, relu=True)

    g = _from_words(gather512b(_to_words(pfs), idxf), 256).reshape(
        VP, L * 256)
    h = _mm(g, _bf(W1a), bias8=_b8(b1a), relu=True)
    g = _from_words(gather512b(_to_words(h), idxf), 256).reshape(VP, L * 256)
    fs = _mm(g, _bf(W1b), bias8=_b8(b1b), relu=True, res=pfs)

    fs, macc = _wd_and_max(fs, _bf(Wd))

    for (Wa, ba, Wb, bb) in ((Wr0a, br0a, Wr0b, br0b),
                             (Wr1a, br1a, Wr1b, br1b),
                             (Wr2a, br2a, Wr2b, br2b)):
        g = lax.bitcast_convert_type(
            gather512b(lax.bitcast_convert_type(fs, jnp.int32), idxf),
            jnp.float32).reshape(VP, L * 128)
        h = _mm(g, _bf(Wa), bias8=_b8(ba), relu=True, out_dtype=jnp.float32)
        g = lax.bitcast_convert_type(
            gather512b(lax.bitcast_convert_type(h, jnp.int32), idxf),
            jnp.float32).reshape(VP, L * 128)
        fs = _mm(g, _bf(Wb), bias8=_b8(bb), relu=True, res=fs,
                 out_dtype=jnp.float32)

    W1p = _bf(Wo1[:256])
    W1f = _bf(Wo1[256:384])
    W1g = _bf(Wo1[384:])
    W3p = _bf(jnp.pad(Wo3, ((0, 0), (0, 128 - Wo3.shape[1]))))
    b3p = jnp.pad(bo3, (0, 128 - bo3.shape[0]))
    o = _final_mlp(pfs, fs, macc, W1p, W1f, W1g, _b8(bo1), _bf(Wo2),
                   _b8(bo2), W3p, _b8(b3p))
    return o[:V, :3].reshape(1, V, 3)


# speedup vs baseline: 14.6298x; 14.6298x over previous
import functools

import jax
import jax.numpy as jnp
from jax import lax
from jax.experimental import pallas as pl
from jax.experimental.pallas import tpu as pltpu
from jax.experimental.pallas import tpu_sc as plsc

V = 10000
VP = 10240
L = 16
NW = 32
GK = 128
NB = 4


def _make_sc_gather(CW: int):
    R = VP * L
    chunk = R // NW
    iters = chunk // GK
    rounds = iters // NB
    rows_per_tile = VP // 16
    mesh = plsc.VectorSubcoreMesh(core_axis_name="c", subcore_axis_name="s")

    @functools.partial(
        pl.kernel,
        mesh=mesh,
        out_type=jax.ShapeDtypeStruct((R, CW), jnp.int32),
        scratch_types=[
            pltpu.VMEM((chunk,), jnp.int32),
        ] + [pltpu.VMEM((GK, CW), jnp.int32) for _ in range(NB)]
          + [pltpu.SemaphoreType.DMA for _ in range(2 * NB)],
    )
    def kfn(table_hbm, idx_hbm, g_hbm, idx_v, *rest):
        bufs = rest[:NB]
        gsems = rest[NB:2 * NB]
        wsems = rest[2 * NB:]
        cid = lax.axis_index("c")
        sid = lax.axis_index("s")
        wid = sid * 2 + cid
        base = pl.multiple_of(wid * chunk, GK)
        pltpu.sync_copy(idx_hbm.at[pl.ds(base, chunk)], idx_v)

        def start_gather(b, g):
            pltpu.async_copy(
                table_hbm.at[idx_v.at[pl.ds(g * GK, GK)]], bufs[b], gsems[b])

        def wait_gather(b, g):
            del g
            pltpu.make_async_copy(
                g_hbm.at[pl.ds(0, GK)], bufs[b], gsems[b]).wait()

        def start_wb(b, g):
            off = pl.multiple_of(base + g * GK, GK)
            pltpu.async_copy(bufs[b], g_hbm.at[pl.ds(off, GK)], wsems[b])

        def wait_wb(b):
            pltpu.make_async_copy(
                bufs[b], g_hbm.at[pl.ds(0, GK)], wsems[b]).wait()

        for b in range(NB):
            start_gather(b, b)

        def body(r, carry):
            for b in range(NB):
                g = r * NB + b
                wait_gather(b, g)
                start_wb(b, g)
            for b in range(NB):
                wait_wb(b)
                start_gather(b, (r + 1) * NB + b)
            return carry

        lax.fori_loop(0, rounds - 1, body, 0)
        for b in range(NB):
            g = (rounds - 1) * NB + b
            wait_gather(b, g)
            start_wb(b, g)
        for b in range(NB):
            wait_wb(b)

    return kfn


def _unpack_words(w):
    wu = w.astype(jnp.uint32)
    lo = (wu & jnp.uint32(0xFFFF)).astype(jnp.uint16)
    hi = (wu >> jnp.uint32(16)).astype(jnp.uint16)
    return (lax.bitcast_convert_type(lo, jnp.bfloat16),
            lax.bitcast_convert_type(hi, jnp.bfloat16))


def _pack_words(yl, yh):
    lo = lax.bitcast_convert_type(
        yl.astype(jnp.bfloat16), jnp.uint16).astype(jnp.uint32)
    hi = lax.bitcast_convert_type(
        yh.astype(jnp.bfloat16), jnp.uint16).astype(jnp.uint32)
    return lax.bitcast_convert_type(lo | (hi << jnp.uint32(16)), jnp.int32)


def _mm(xp, Ws, bias8=None, relu=False, res=None, mask=True,
        unpack_in=False, out=("val",), out_dtype=jnp.bfloat16, BM=256):
    M, K = xp.shape
    N = Ws[0].shape[1]
    nblk = M // BM
    args = [xp] + list(Ws)
    in_specs = [pl.BlockSpec((BM, K), lambda i: (i, 0))] + [
        pl.BlockSpec(W.shape, lambda i: (0, 0)) for W in Ws]
    if bias8 is not None:
        args.append(bias8)
        in_specs.append(pl.BlockSpec((8, N), lambda i: (0, 0)))
    if res is not None:
        args.append(res)
        in_specs.append(pl.BlockSpec((BM, N), lambda i: (i, 0)))
    have_bias = bias8 is not None
    have_res = res is not None
    nw = len(Ws)

    def body(*refs):
        x_ref = refs[0]
        w_refs = refs[1:1 + nw]
        rest = refs[1 + nw:-len(out)]
        o_refs = refs[-len(out):]
        if unpack_in:
            lo, hi = _unpack_words(x_ref[...])
            y = jnp.dot(lo, w_refs[0][...], preferred_element_type=jnp.float32)
            y = y + jnp.dot(hi, w_refs[1][...],
                            preferred_element_type=jnp.float32)
        else:
            y = jnp.dot(x_ref[...].astype(jnp.bfloat16), w_refs[0][...],
                        preferred_element_type=jnp.float32)
        ri = 0
        if have_bias:
            y = y + rest[0][0:1, :]
            ri = 1
        if mask:
            i = pl.program_id(0)
            rows = i * BM + lax.broadcasted_iota(jnp.int32, (BM, 1), 0)
            y = jnp.where(rows < V, y, 0.0)
        if have_res:
            y = y + rest[ri][...].astype(jnp.float32)
        if relu:
            y = jnp.maximum(y, 0.0)
        for kind, o_ref in zip(out, o_refs):
            if kind == "pack":
                o_ref[...] = _pack_words(y[:, :N // 2], y[:, N // 2:])
            else:
                o_ref[...] = y.astype(o_ref.dtype)

    out_specs = []
    out_shapes = []
    for kind in out:
        if kind == "pack":
            out_specs.append(pl.BlockSpec((BM, N // 2), lambda i: (i, 0)))
            out_shapes.append(jax.ShapeDtypeStruct((M, N // 2), jnp.int32))
        else:
            out_specs.append(pl.BlockSpec((BM, N), lambda i: (i, 0)))
            out_shapes.append(jax.ShapeDtypeStruct((M, N), out_dtype))
    r = pl.pallas_call(
        body,
        grid=(nblk,),
        in_specs=in_specs,
        out_specs=out_specs if len(out) > 1 else out_specs[0],
        out_shape=out_shapes if len(out) > 1 else out_shapes[0],
    )(*args)
    return r


def _wd_and_max(fs, Wd, BM=256):
    M, K = fs.shape
    N = Wd.shape[1]
    nblk = M // BM

    def body(x_ref, w_ref, o_ref, m_ref):
        i = pl.program_id(0)
        y = jnp.dot(x_ref[...], w_ref[...], preferred_element_type=jnp.float32)
        rows = i * BM + lax.broadcasted_iota(jnp.int32, (BM, 1), 0)
        valid = rows < V
        yv = jnp.where(valid, y, 0.0)
        o_ref[...] = yv
        ym = jnp.where(valid, y, -1e30)
        m = ym[0:8]
        for j in range(1, BM // 8):
            m = jnp.maximum(m, ym[j * 8:(j + 1) * 8])

        @pl.when(i == 0)
        def _():
            m_ref[...] = m

        @pl.when(i > 0)
        def _():
            m_ref[...] = jnp.maximum(m_ref[...], m)

    return pl.pallas_call(
        body,
        grid=(nblk,),
        in_specs=[
            pl.BlockSpec((BM, K), lambda i: (i, 0)),
            pl.BlockSpec((K, N), lambda i: (0, 0)),
        ],
        out_specs=[
            pl.BlockSpec((BM, N), lambda i: (i, 0)),
            pl.BlockSpec((8, N), lambda i: (0, 0)),
        ],
        out_shape=[
            jax.ShapeDtypeStruct((M, N), jnp.float32),
            jax.ShapeDtypeStruct((8, N), jnp.float32),
        ],
    )(fs, Wd)


def _final_mlp(pfs, fs, macc, W1p, W1f, W1g, b1_8, W2, b2_8, W3p, b3_8,
               BM=256):
    M = pfs.shape[0]
    nblk = M // BM

    def body(p_ref, f_ref, g_ref, w1p, w1f, w1g, b1, w2, b2, w3, b3, o_ref):
        gmax = jnp.max(g_ref[...], axis=0, keepdims=True)
        gc = jnp.dot(gmax.astype(jnp.bfloat16), w1g[...],
                     preferred_element_type=jnp.float32)
        o1 = jnp.dot(p_ref[...], w1p[...], preferred_element_type=jnp.float32)
        o1 = o1 + jnp.dot(f_ref[...].astype(jnp.bfloat16), w1f[...],
                          preferred_element_type=jnp.float32)
        o1 = jnp.maximum(o1 + gc + b1[0:1, :], 0.0).astype(jnp.bfloat16)
        o2 = jnp.maximum(
            jnp.dot(o1, w2[...], preferred_element_type=jnp.float32)
            + b2[0:1, :], 0.0).astype(jnp.bfloat16)
        o_ref[...] = (
            jnp.dot(o2, w3[...], preferred_element_type=jnp.float32)
            + b3[0:1, :])

    specs = [
        pl.BlockSpec((BM, 256), lambda i: (i, 0)),
        pl.BlockSpec((BM, 128), lambda i: (i, 0)),
        pl.BlockSpec((8, 128), lambda i: (0, 0)),
        pl.BlockSpec((256, 256), lambda i: (0, 0)),
        pl.BlockSpec((128, 256), lambda i: (0, 0)),
        pl.BlockSpec((128, 256), lambda i: (0, 0)),
        pl.BlockSpec((8, 256), lambda i: (0, 0)),
        pl.BlockSpec((256, 128), lambda i: (0, 0)),
        pl.BlockSpec((8, 128), lambda i: (0, 0)),
        pl.BlockSpec((128, 128), lambda i: (0, 0)),
        pl.BlockSpec((8, 128), lambda i: (0, 0)),
    ]
    return pl.pallas_call(
        body,
        grid=(nblk,),
        in_specs=specs,
        out_specs=pl.BlockSpec((BM, 128), lambda i: (i, 0)),
        out_shape=jax.ShapeDtypeStruct((M, 128), jnp.float32),
    )(pfs, fs, macc, W1p, W1f, W1g, b1_8, W2, b2_8, W3p, b3_8)


def _b8(b):
    return jnp.broadcast_to(b.reshape(1, -1), (8, b.shape[0]))


def _bf(a):
    return a.astype(jnp.bfloat16)


def kernel(x, spiral, Wp, W1a, b1a, W1b, b1b, Wd, Wr0a, br0a, Wr0b, br0b,
           Wr1a, br1a, Wr1b, br1b, Wr2a, br2a, Wr2b, br2b, Wo1, bo1, Wo2,
           bo2, Wo3, bo3):
    Bn, Vn, FIN = x.shape
    KP = 512
    xp = _bf(jnp.pad(x[0], ((0, VP - Vn), (0, KP - FIN))))
    Wpp = _bf(jnp.pad(Wp, ((0, KP - FIN), (0, 0))))
    idxf = jnp.pad(spiral.reshape(-1), (0, VP * L - spiral.size))
    idxf = idxf.astype(jnp.int32)

    gather512b = _make_sc_gather(128)

    def lohi(W):
        Wr = W.reshape(L, 256, W.shape[1])
        return [_bf(Wr[:, :128].reshape(L * 128, -1)),
                _bf(Wr[:, 128:].reshape(L * 128, -1))]

    pfs_w, pfs = _mm(xp, [Wpp], relu=True, out=("pack", "val"))

    g = gather512b(pfs_w, idxf).reshape(VP, L * 128)
    h_w = _mm(g, lohi(W1a), bias8=_b8(b1a), relu=True, unpack_in=True,
              out=("pack",))
    g = gather512b(h_w, idxf).reshape(VP, L * 128)
    fs = _mm(g, lohi(W1b), bias8=_b8(b1b), relu=True, res=pfs,
             unpack_in=True)

    fs, macc = _wd_and_max(fs, _bf(Wd))

    for (Wa, ba, Wb, bb) in ((Wr0a, br0a, Wr0b, br0b),
                             (Wr1a, br1a, Wr1b, br1b),
                             (Wr2a, br2a, Wr2b, br2b)):
        g = lax.bitcast_convert_type(
            gather512b(lax.bitcast_convert_type(fs, jnp.int32), idxf),
            jnp.float32).reshape(VP, L * 128)
        h = _mm(g, [_bf(Wa)], bias8=_b8(ba), relu=True,
                out_dtype=jnp.float32)
        g = lax.bitcast_convert_type(
            gather512b(lax.bitcast_convert_type(h, jnp.int32), idxf),
            jnp.float32).reshape(VP, L * 128)
        fs = _mm(g, [_bf(Wb)], bias8=_b8(bb), relu=True, res=fs,
                 out_dtype=jnp.float32)

    W1p = _bf(Wo1[:256])
    W1f = _bf(Wo1[256:384])
    W1g = _bf(Wo1[384:])
    W3p = _bf(jnp.pad(Wo3, ((0, 0), (0, 128 - Wo3.shape[1]))))
    b3p = jnp.pad(bo3, (0, 128 - bo3.shape[0]))
    o = _final_mlp(pfs, fs, macc, W1p, W1f, W1g, _b8(bo1), _bf(Wo2),
                   _b8(bo2), W3p, _b8(b3p))
    return o[:V, :3].reshape(1, V, 3)


# speedup vs baseline: 14.6792x; 1.0034x over previous
import functools

import jax
import jax.numpy as jnp
from jax import lax
from jax.experimental import pallas as pl
from jax.experimental.pallas import tpu as pltpu
from jax.experimental.pallas import tpu_sc as plsc

V = 10000
VP = 10240
L = 16
NW = 32
GK = 128
NB = 5


def _make_sc_gather(CW: int):
    R = VP * L
    chunk = R // NW
    iters = chunk // GK
    rounds = iters // NB
    rows_per_tile = VP // 16
    mesh = plsc.VectorSubcoreMesh(core_axis_name="c", subcore_axis_name="s")

    @functools.partial(
        pl.kernel,
        mesh=mesh,
        out_type=jax.ShapeDtypeStruct((R, CW), jnp.int32),
        scratch_types=[
            pltpu.VMEM((chunk,), jnp.int32),
        ] + [pltpu.VMEM((GK, CW), jnp.int32) for _ in range(NB)]
          + [pltpu.SemaphoreType.DMA for _ in range(2 * NB)],
    )
    def kfn(table_hbm, idx_hbm, g_hbm, idx_v, *rest):
        bufs = rest[:NB]
        gsems = rest[NB:2 * NB]
        wsems = rest[2 * NB:]
        cid = lax.axis_index("c")
        sid = lax.axis_index("s")
        wid = sid * 2 + cid
        base = pl.multiple_of(wid * chunk, GK)
        pltpu.sync_copy(idx_hbm.at[pl.ds(base, chunk)], idx_v)

        def start_gather(b, g):
            pltpu.async_copy(
                table_hbm.at[idx_v.at[pl.ds(g * GK, GK)]], bufs[b], gsems[b])

        def wait_gather(b, g):
            del g
            pltpu.make_async_copy(
                g_hbm.at[pl.ds(0, GK)], bufs[b], gsems[b]).wait()

        def start_wb(b, g):
            off = pl.multiple_of(base + g * GK, GK)
            pltpu.async_copy(bufs[b], g_hbm.at[pl.ds(off, GK)], wsems[b])

        def wait_wb(b):
            pltpu.make_async_copy(
                bufs[b], g_hbm.at[pl.ds(0, GK)], wsems[b]).wait()

        for b in range(NB):
            start_gather(b, b)

        def body(r, carry):
            for b in range(NB):
                g = r * NB + b
                wait_gather(b, g)
                start_wb(b, g)
            for b in range(NB):
                wait_wb(b)
                start_gather(b, (r + 1) * NB + b)
            return carry

        lax.fori_loop(0, rounds - 1, body, 0)
        for b in range(NB):
            g = (rounds - 1) * NB + b
            wait_gather(b, g)
            start_wb(b, g)
        for b in range(NB):
            wait_wb(b)

    return kfn


def _unpack_words(w):
    wu = w.astype(jnp.uint32)
    lo = (wu & jnp.uint32(0xFFFF)).astype(jnp.uint16)
    hi = (wu >> jnp.uint32(16)).astype(jnp.uint16)
    return (lax.bitcast_convert_type(lo, jnp.bfloat16),
            lax.bitcast_convert_type(hi, jnp.bfloat16))


def _pack_words(yl, yh):
    lo = lax.bitcast_convert_type(
        yl.astype(jnp.bfloat16), jnp.uint16).astype(jnp.uint32)
    hi = lax.bitcast_convert_type(
        yh.astype(jnp.bfloat16), jnp.uint16).astype(jnp.uint32)
    return lax.bitcast_convert_type(lo | (hi << jnp.uint32(16)), jnp.int32)


def _mm(xp, Ws, bias8=None, relu=False, res=None, mask=True,
        unpack_in=False, out=("val",), out_dtype=jnp.bfloat16, BM=256):
    M, K = xp.shape
    N = Ws[0].shape[1]
    nblk = M // BM
    args = [xp] + list(Ws)
    in_specs = [pl.BlockSpec((BM, K), lambda i: (i, 0))] + [
        pl.BlockSpec(W.shape, lambda i: (0, 0)) for W in Ws]
    if bias8 is not None:
        args.append(bias8)
        in_specs.append(pl.BlockSpec((8, N), lambda i: (0, 0)))
    if res is not None:
        args.append(res)
        in_specs.append(pl.BlockSpec((BM, N), lambda i: (i, 0)))
    have_bias = bias8 is not None
    have_res = res is not None
    nw = len(Ws)

    def body(*refs):
        x_ref = refs[0]
        w_refs = refs[1:1 + nw]
        rest = refs[1 + nw:-len(out)]
        o_refs = refs[-len(out):]
        if unpack_in:
            lo, hi = _unpack_words(x_ref[...])
            y = jnp.dot(lo, w_refs[0][...], preferred_element_type=jnp.float32)
            y = y + jnp.dot(hi, w_refs[1][...],
                            preferred_element_type=jnp.float32)
        else:
            y = jnp.dot(x_ref[...].astype(jnp.bfloat16), w_refs[0][...],
                        preferred_element_type=jnp.float32)
        ri = 0
        if have_bias:
            y = y + rest[0][0:1, :]
            ri = 1
        if mask:
            i = pl.program_id(0)
            rows = i * BM + lax.broadcasted_iota(jnp.int32, (BM, 1), 0)
            y = jnp.where(rows < V, y, 0.0)
        if have_res:
            y = y + rest[ri][...].astype(jnp.float32)
        if relu:
            y = jnp.maximum(y, 0.0)
        for kind, o_ref in zip(out, o_refs):
            if kind == "pack":
                o_ref[...] = _pack_words(y[:, :N // 2], y[:, N // 2:])
            else:
                o_ref[...] = y.astype(o_ref.dtype)

    out_specs = []
    out_shapes = []
    for kind in out:
        if kind == "pack":
            out_specs.append(pl.BlockSpec((BM, N // 2), lambda i: (i, 0)))
            out_shapes.append(jax.ShapeDtypeStruct((M, N // 2), jnp.int32))
        else:
            out_specs.append(pl.BlockSpec((BM, N), lambda i: (i, 0)))
            out_shapes.append(jax.ShapeDtypeStruct((M, N), out_dtype))
    r = pl.pallas_call(
        body,
        grid=(nblk,),
        in_specs=in_specs,
        out_specs=out_specs if len(out) > 1 else out_specs[0],
        out_shape=out_shapes if len(out) > 1 else out_shapes[0],
    )(*args)
    return r


def _wd_and_max(fs, Wd, BM=256):
    M, K = fs.shape
    N = Wd.shape[1]
    nblk = M // BM

    def body(x_ref, w_ref, o_ref, m_ref):
        i = pl.program_id(0)
        y = jnp.dot(x_ref[...], w_ref[...], preferred_element_type=jnp.float32)
        rows = i * BM + lax.broadcasted_iota(jnp.int32, (BM, 1), 0)
        valid = rows < V
        yv = jnp.where(valid, y, 0.0)
        o_ref[...] = yv
        ym = jnp.where(valid, y, -1e30)
        m = ym[0:8]
        for j in range(1, BM // 8):
            m = jnp.maximum(m, ym[j * 8:(j + 1) * 8])

        @pl.when(i == 0)
        def _():
            m_ref[...] = m

        @pl.when(i > 0)
        def _():
            m_ref[...] = jnp.maximum(m_ref[...], m)

    return pl.pallas_call(
        body,
        grid=(nblk,),
        in_specs=[
            pl.BlockSpec((BM, K), lambda i: (i, 0)),
            pl.BlockSpec((K, N), lambda i: (0, 0)),
        ],
        out_specs=[
            pl.BlockSpec((BM, N), lambda i: (i, 0)),
            pl.BlockSpec((8, N), lambda i: (0, 0)),
        ],
        out_shape=[
            jax.ShapeDtypeStruct((M, N), jnp.float32),
            jax.ShapeDtypeStruct((8, N), jnp.float32),
        ],
    )(fs, Wd)


def _final_mlp(pfs, fs, macc, W1p, W1f, W1g, b1_8, W2, b2_8, W3p, b3_8,
               BM=256):
    M = pfs.shape[0]
    nblk = M // BM

    def body(p_ref, f_ref, g_ref, w1p, w1f, w1g, b1, w2, b2, w3, b3, o_ref):
        gmax = jnp.max(g_ref[...], axis=0, keepdims=True)
        gc = jnp.dot(gmax.astype(jnp.bfloat16), w1g[...],
                     preferred_element_type=jnp.float32)
        o1 = jnp.dot(p_ref[...], w1p[...], preferred_element_type=jnp.float32)
        o1 = o1 + jnp.dot(f_ref[...].astype(jnp.bfloat16), w1f[...],
                          preferred_element_type=jnp.float32)
        o1 = jnp.maximum(o1 + gc + b1[0:1, :], 0.0).astype(jnp.bfloat16)
        o2 = jnp.maximum(
            jnp.dot(o1, w2[...], preferred_element_type=jnp.float32)
            + b2[0:1, :], 0.0).astype(jnp.bfloat16)
        o_ref[...] = (
            jnp.dot(o2, w3[...], preferred_element_type=jnp.float32)
            + b3[0:1, :])

    specs = [
        pl.BlockSpec((BM, 256), lambda i: (i, 0)),
        pl.BlockSpec((BM, 128), lambda i: (i, 0)),
        pl.BlockSpec((8, 128), lambda i: (0, 0)),
        pl.BlockSpec((256, 256), lambda i: (0, 0)),
        pl.BlockSpec((128, 256), lambda i: (0, 0)),
        pl.BlockSpec((128, 256), lambda i: (0, 0)),
        pl.BlockSpec((8, 256), lambda i: (0, 0)),
        pl.BlockSpec((256, 128), lambda i: (0, 0)),
        pl.BlockSpec((8, 128), lambda i: (0, 0)),
        pl.BlockSpec((128, 128), lambda i: (0, 0)),
        pl.BlockSpec((8, 128), lambda i: (0, 0)),
    ]
    return pl.pallas_call(
        body,
        grid=(nblk,),
        in_specs=specs,
        out_specs=pl.BlockSpec((BM, 128), lambda i: (i, 0)),
        out_shape=jax.ShapeDtypeStruct((M, 128), jnp.float32),
    )(pfs, fs, macc, W1p, W1f, W1g, b1_8, W2, b2_8, W3p, b3_8)


def _b8(b):
    return jnp.broadcast_to(b.reshape(1, -1), (8, b.shape[0]))


def _bf(a):
    return a.astype(jnp.bfloat16)


def kernel(x, spiral, Wp, W1a, b1a, W1b, b1b, Wd, Wr0a, br0a, Wr0b, br0b,
           Wr1a, br1a, Wr1b, br1b, Wr2a, br2a, Wr2b, br2b, Wo1, bo1, Wo2,
           bo2, Wo3, bo3):
    Bn, Vn, FIN = x.shape
    KP = 512
    xp = _bf(jnp.pad(x[0], ((0, VP - Vn), (0, KP - FIN))))
    Wpp = _bf(jnp.pad(Wp, ((0, KP - FIN), (0, 0))))
    idxf = jnp.pad(spiral.reshape(-1), (0, VP * L - spiral.size))
    idxf = idxf.astype(jnp.int32)

    gather512b = _make_sc_gather(128)

    def lohi(W):
        Wr = W.reshape(L, 256, W.shape[1])
        return [_bf(Wr[:, :128].reshape(L * 128, -1)),
                _bf(Wr[:, 128:].reshape(L * 128, -1))]

    pfs_w, pfs = _mm(xp, [Wpp], relu=True, out=("pack", "val"))

    g = gather512b(pfs_w, idxf).reshape(VP, L * 128)
    h_w = _mm(g, lohi(W1a), bias8=_b8(b1a), relu=True, unpack_in=True,
              out=("pack",))
    g = gather512b(h_w, idxf).reshape(VP, L * 128)
    fs = _mm(g, lohi(W1b), bias8=_b8(b1b), relu=True, res=pfs,
             unpack_in=True)

    fs, macc = _wd_and_max(fs, _bf(Wd))

    for (Wa, ba, Wb, bb) in ((Wr0a, br0a, Wr0b, br0b),
                             (Wr1a, br1a, Wr1b, br1b),
                             (Wr2a, br2a, Wr2b, br2b)):
        g = lax.bitcast_convert_type(
            gather512b(lax.bitcast_convert_type(fs, jnp.int32), idxf),
            jnp.float32).reshape(VP, L * 128)
        h = _mm(g, [_bf(Wa)], bias8=_b8(ba), relu=True,
                out_dtype=jnp.float32)
        g = lax.bitcast_convert_type(
            gather512b(lax.bitcast_convert_type(h, jnp.int32), idxf),
            jnp.float32).reshape(VP, L * 128)
        fs = _mm(g, [_bf(Wb)], bias8=_b8(bb), relu=True, res=fs,
                 out_dtype=jnp.float32)

    W1p = _bf(Wo1[:256])
    W1f = _bf(Wo1[256:384])
    W1g = _bf(Wo1[384:])
    W3p = _bf(jnp.pad(Wo3, ((0, 0), (0, 128 - Wo3.shape[1]))))
    b3p = jnp.pad(bo3, (0, 128 - bo3.shape[0]))
    o = _final_mlp(pfs, fs, macc, W1p, W1f, W1g, _b8(bo1), _bf(Wo2),
                   _b8(bo2), W3p, _b8(b3p))
    return o[:V, :3].reshape(1, V, 3)


# speedup vs baseline: 15.7525x; 1.0731x over previous
import functools

import jax
import jax.numpy as jnp
from jax import lax
from jax.experimental import pallas as pl
from jax.experimental.pallas import tpu as pltpu
from jax.experimental.pallas import tpu_sc as plsc

V = 10000
VP = 10240
L = 16
NW = 32
GK = 128
NB = 5


def _make_sc_gather(CW: int, nch: int = 1):
    R = VP * L // nch
    chunk = R // NW
    iters = chunk // GK
    rounds = iters // NB
    rows_per_tile = VP // 16
    mesh = plsc.VectorSubcoreMesh(core_axis_name="c", subcore_axis_name="s")

    @functools.partial(
        pl.kernel,
        mesh=mesh,
        out_type=jax.ShapeDtypeStruct((R, CW), jnp.int32),
        scratch_types=[
            pltpu.VMEM((chunk,), jnp.int32),
        ] + [pltpu.VMEM((GK, CW), jnp.int32) for _ in range(NB)]
          + [pltpu.SemaphoreType.DMA for _ in range(2 * NB)],
    )
    def kfn(table_hbm, idx_hbm, g_hbm, idx_v, *rest):
        bufs = rest[:NB]
        gsems = rest[NB:2 * NB]
        wsems = rest[2 * NB:]
        cid = lax.axis_index("c")
        sid = lax.axis_index("s")
        wid = sid * 2 + cid
        base = pl.multiple_of(wid * chunk, GK)
        pltpu.sync_copy(idx_hbm.at[pl.ds(base, chunk)], idx_v)

        def start_gather(b, g):
            pltpu.async_copy(
                table_hbm.at[idx_v.at[pl.ds(g * GK, GK)]], bufs[b], gsems[b])

        def wait_gather(b, g):
            del g
            pltpu.make_async_copy(
                g_hbm.at[pl.ds(0, GK)], bufs[b], gsems[b]).wait()

        def start_wb(b, g):
            off = pl.multiple_of(base + g * GK, GK)
            pltpu.async_copy(bufs[b], g_hbm.at[pl.ds(off, GK)], wsems[b])

        def wait_wb(b):
            pltpu.make_async_copy(
                bufs[b], g_hbm.at[pl.ds(0, GK)], wsems[b]).wait()

        for b in range(NB):
            start_gather(b, b)

        def body(r, carry):
            for b in range(NB):
                g = r * NB + b
                wait_gather(b, g)
                start_wb(b, g)
            for b in range(NB):
                wait_wb(b)
                start_gather(b, (r + 1) * NB + b)
            return carry

        lax.fori_loop(0, rounds - 1, body, 0)
        for b in range(NB):
            g = (rounds - 1) * NB + b
            wait_gather(b, g)
            start_wb(b, g)
        for b in range(NB):
            wait_wb(b)

    return kfn


def _unpack_words(w):
    wu = w.astype(jnp.uint32)
    lo = (wu & jnp.uint32(0xFFFF)).astype(jnp.uint16)
    hi = (wu >> jnp.uint32(16)).astype(jnp.uint16)
    return (lax.bitcast_convert_type(lo, jnp.bfloat16),
            lax.bitcast_convert_type(hi, jnp.bfloat16))


def _pack_words(yl, yh):
    lo = lax.bitcast_convert_type(
        yl.astype(jnp.bfloat16), jnp.uint16).astype(jnp.uint32)
    hi = lax.bitcast_convert_type(
        yh.astype(jnp.bfloat16), jnp.uint16).astype(jnp.uint32)
    return lax.bitcast_convert_type(lo | (hi << jnp.uint32(16)), jnp.int32)


def _mm(xp, Ws, bias8=None, relu=False, res=None, mask=True, row_off=0,
        unpack_in=False, out=("val",), out_dtype=jnp.bfloat16, BM=256):
    M, K = xp.shape
    N = Ws[0].shape[1]
    nblk = M // BM
    args = [xp] + list(Ws)
    in_specs = [pl.BlockSpec((BM, K), lambda i: (i, 0))] + [
        pl.BlockSpec(W.shape, lambda i: (0, 0)) for W in Ws]
    if bias8 is not None:
        args.append(bias8)
        in_specs.append(pl.BlockSpec((8, N), lambda i: (0, 0)))
    if res is not None:
        args.append(res)
        in_specs.append(pl.BlockSpec((BM, N), lambda i: (i, 0)))
    have_bias = bias8 is not None
    have_res = res is not None
    nw = len(Ws)

    def body(*refs):
        x_ref = refs[0]
        w_refs = refs[1:1 + nw]
        rest = refs[1 + nw:-len(out)]
        o_refs = refs[-len(out):]
        if unpack_in:
            lo, hi = _unpack_words(x_ref[...])
            y = jnp.dot(lo, w_refs[0][...], preferred_element_type=jnp.float32)
            y = y + jnp.dot(hi, w_refs[1][...],
                            preferred_element_type=jnp.float32)
        else:
            y = jnp.dot(x_ref[...].astype(jnp.bfloat16), w_refs[0][...],
                        preferred_element_type=jnp.float32)
        ri = 0
        if have_bias:
            y = y + rest[0][0:1, :]
            ri = 1
        if mask:
            i = pl.program_id(0)
            rows = row_off + i * BM + lax.broadcasted_iota(
                jnp.int32, (BM, 1), 0)
            y = jnp.where(rows < V, y, 0.0)
        if have_res:
            y = y + rest[ri][...].astype(jnp.float32)
        if relu:
            y = jnp.maximum(y, 0.0)
        for kind, o_ref in zip(out, o_refs):
            if kind == "pack":
                o_ref[...] = _pack_words(y[:, :N // 2], y[:, N // 2:])
            else:
                o_ref[...] = y.astype(o_ref.dtype)

    out_specs = []
    out_shapes = []
    for kind in out:
        if kind == "pack":
            out_specs.append(pl.BlockSpec((BM, N // 2), lambda i: (i, 0)))
            out_shapes.append(jax.ShapeDtypeStruct((M, N // 2), jnp.int32))
        else:
            out_specs.append(pl.BlockSpec((BM, N), lambda i: (i, 0)))
            out_shapes.append(jax.ShapeDtypeStruct((M, N), out_dtype))
    r = pl.pallas_call(
        body,
        grid=(nblk,),
        in_specs=in_specs,
        out_specs=out_specs if len(out) > 1 else out_specs[0],
        out_shape=out_shapes if len(out) > 1 else out_shapes[0],
    )(*args)
    return r


def _wd_and_max(fs, Wd, BM=256):
    M, K = fs.shape
    N = Wd.shape[1]
    nblk = M // BM

    def body(x_ref, w_ref, o_ref, m_ref):
        i = pl.program_id(0)
        y = jnp.dot(x_ref[...], w_ref[...], preferred_element_type=jnp.float32)
        rows = i * BM + lax.broadcasted_iota(jnp.int32, (BM, 1), 0)
        valid = rows < V
        yv = jnp.where(valid, y, 0.0)
        o_ref[...] = yv
        ym = jnp.where(valid, y, -1e30)
        m = ym[0:8]
        for j in range(1, BM // 8):
            m = jnp.maximum(m, ym[j * 8:(j + 1) * 8])

        @pl.when(i == 0)
        def _():
            m_ref[...] = m

        @pl.when(i > 0)
        def _():
            m_ref[...] = jnp.maximum(m_ref[...], m)

    return pl.pallas_call(
        body,
        grid=(nblk,),
        in_specs=[
            pl.BlockSpec((BM, K), lambda i: (i, 0)),
            pl.BlockSpec((K, N), lambda i: (0, 0)),
        ],
        out_specs=[
            pl.BlockSpec((BM, N), lambda i: (i, 0)),
            pl.BlockSpec((8, N), lambda i: (0, 0)),
        ],
        out_shape=[
            jax.ShapeDtypeStruct((M, N), jnp.float32),
            jax.ShapeDtypeStruct((8, N), jnp.float32),
        ],
    )(fs, Wd)


def _final_mlp(pfs, fs, macc, W1p, W1f, W1g, b1_8, W2, b2_8, W3p, b3_8,
               BM=256):
    M = pfs.shape[0]
    nblk = M // BM

    def body(p_ref, f_ref, g_ref, w1p, w1f, w1g, b1, w2, b2, w3, b3, o_ref):
        gmax = jnp.max(g_ref[...], axis=0, keepdims=True)
        gc = jnp.dot(gmax.astype(jnp.bfloat16), w1g[...],
                     preferred_element_type=jnp.float32)
        o1 = jnp.dot(p_ref[...], w1p[...], preferred_element_type=jnp.float32)
        o1 = o1 + jnp.dot(f_ref[...].astype(jnp.bfloat16), w1f[...],
                          preferred_element_type=jnp.float32)
        o1 = jnp.maximum(o1 + gc + b1[0:1, :], 0.0).astype(jnp.bfloat16)
        o2 = jnp.maximum(
            jnp.dot(o1, w2[...], preferred_element_type=jnp.float32)
            + b2[0:1, :], 0.0).astype(jnp.bfloat16)
        o_ref[...] = (
            jnp.dot(o2, w3[...], preferred_element_type=jnp.float32)
            + b3[0:1, :])

    specs = [
        pl.BlockSpec((BM, 256), lambda i: (i, 0)),
        pl.BlockSpec((BM, 128), lambda i: (i, 0)),
        pl.BlockSpec((8, 128), lambda i: (0, 0)),
        pl.BlockSpec((256, 256), lambda i: (0, 0)),
        pl.BlockSpec((128, 256), lambda i: (0, 0)),
        pl.BlockSpec((128, 256), lambda i: (0, 0)),
        pl.BlockSpec((8, 256), lambda i: (0, 0)),
        pl.BlockSpec((256, 128), lambda i: (0, 0)),
        pl.BlockSpec((8, 128), lambda i: (0, 0)),
        pl.BlockSpec((128, 128), lambda i: (0, 0)),
        pl.BlockSpec((8, 128), lambda i: (0, 0)),
    ]
    return pl.pallas_call(
        body,
        grid=(nblk,),
        in_specs=specs,
        out_specs=pl.BlockSpec((BM, 128), lambda i: (i, 0)),
        out_shape=jax.ShapeDtypeStruct((M, 128), jnp.float32),
    )(pfs, fs, macc, W1p, W1f, W1g, b1_8, W2, b2_8, W3p, b3_8)


def _b8(b):
    return jnp.broadcast_to(b.reshape(1, -1), (8, b.shape[0]))


def _bf(a):
    return a.astype(jnp.bfloat16)


def kernel(x, spiral, Wp, W1a, b1a, W1b, b1b, Wd, Wr0a, br0a, Wr0b, br0b,
           Wr1a, br1a, Wr1b, br1b, Wr2a, br2a, Wr2b, br2b, Wo1, bo1, Wo2,
           bo2, Wo3, bo3):
    Bn, Vn, FIN = x.shape
    KP = 512
    xp = _bf(jnp.pad(x[0], ((0, VP - Vn), (0, KP - FIN))))
    Wpp = _bf(jnp.pad(Wp, ((0, KP - FIN), (0, 0))))
    idxf = jnp.pad(spiral.reshape(-1), (0, VP * L - spiral.size))
    idxf = idxf.astype(jnp.int32)

    NCH = 2
    VPc = VP // NCH
    Rc = VP * L // NCH
    gather512b = _make_sc_gather(128, NCH)
    idxc = [idxf[c * Rc:(c + 1) * Rc] for c in range(NCH)]

    def spiral_conv(tbl_words, Ws, bias8, unpack, res=None, relu=True,
                    out=("val",), out_dtype=jnp.bfloat16):
        gs = [gather512b(tbl_words, idxc[c]) for c in range(NCH)]
        K = gs[0].shape[1] * L
        outs = []
        for c in range(NCH):
            rsl = None if res is None else res[c * VPc:(c + 1) * VPc]
            xin = gs[c] if unpack else lax.bitcast_convert_type(
                gs[c], jnp.float32)
            outs.append(_mm(xin.reshape(VPc, K), Ws, bias8=bias8,
                            relu=relu, res=rsl, row_off=c * VPc,
                            unpack_in=unpack, out=out, out_dtype=out_dtype))
        if len(out) > 1:
            return [jnp.concatenate([o[j] for o in outs])
                    for j in range(len(out))]
        return jnp.concatenate(outs)

    def lohi(W):
        Wr = W.reshape(L, 256, W.shape[1])
        return [_bf(Wr[:, :128].reshape(L * 128, -1)),
                _bf(Wr[:, 128:].reshape(L * 128, -1))]

    pfs_w, pfs = _mm(xp, [Wpp], relu=True, out=("pack", "val"))

    h_w = spiral_conv(pfs_w, lohi(W1a), _b8(b1a), True, out=("pack",))
    fs = spiral_conv(h_w, lohi(W1b), _b8(b1b), True, res=pfs)

    fs, macc = _wd_and_max(fs, _bf(Wd))

    for (Wa, ba, Wb, bb) in ((Wr0a, br0a, Wr0b, br0b),
                             (Wr1a, br1a, Wr1b, br1b),
                             (Wr2a, br2a, Wr2b, br2b)):
        h = spiral_conv(lax.bitcast_convert_type(fs, jnp.int32),
                        [_bf(Wa)], _b8(ba), False, out_dtype=jnp.float32)
        fs = spiral_conv(lax.bitcast_convert_type(h, jnp.int32),
                         [_bf(Wb)], _b8(bb), False, res=fs,
                         out_dtype=jnp.float32)

    W1p = _bf(Wo1[:256])
    W1f = _bf(Wo1[256:384])
    W1g = _bf(Wo1[384:])
    W3p = _bf(jnp.pad(Wo3, ((0, 0), (0, 128 - Wo3.shape[1]))))
    b3p = jnp.pad(bo3, (0, 128 - bo3.shape[0]))
    o = _final_mlp(pfs, fs, macc, W1p, W1f, W1g, _b8(bo1), _bf(Wo2),
                   _b8(bo2), W3p, _b8(b3p))
    return o[:V, :3].reshape(1, V, 3)


# speedup vs baseline: 15.9360x; 1.0117x over previous
import functools

import jax
import jax.numpy as jnp
from jax import lax
from jax.experimental import pallas as pl
from jax.experimental.pallas import tpu as pltpu
from jax.experimental.pallas import tpu_sc as plsc

V = 10000
VP = 10240
L = 16
NW = 32
GK = 128
NB = 5


def _make_sc_gather(CW: int, nch: int = 1):
    R = VP * L // nch
    chunk = R // NW
    iters = chunk // GK
    rounds = iters // NB
    rows_per_tile = VP // 16
    mesh = plsc.VectorSubcoreMesh(core_axis_name="c", subcore_axis_name="s")

    @functools.partial(
        pl.kernel,
        mesh=mesh,
        out_type=jax.ShapeDtypeStruct((R, CW), jnp.int32),
        scratch_types=[
            pltpu.VMEM((chunk,), jnp.int32),
        ] + [pltpu.VMEM((GK, CW), jnp.int32) for _ in range(NB)]
          + [pltpu.SemaphoreType.DMA for _ in range(2 * NB)],
    )
    def kfn(table_hbm, idx_hbm, g_hbm, idx_v, *rest):
        bufs = rest[:NB]
        gsems = rest[NB:2 * NB]
        wsems = rest[2 * NB:]
        cid = lax.axis_index("c")
        sid = lax.axis_index("s")
        wid = sid * 2 + cid
        base = pl.multiple_of(wid * chunk, GK)
        pltpu.sync_copy(idx_hbm.at[pl.ds(base, chunk)], idx_v)

        def start_gather(b, g):
            pltpu.async_copy(
                table_hbm.at[idx_v.at[pl.ds(g * GK, GK)]], bufs[b], gsems[b])

        def wait_gather(b, g):
            del g
            pltpu.make_async_copy(
                g_hbm.at[pl.ds(0, GK)], bufs[b], gsems[b]).wait()

        def start_wb(b, g):
            off = pl.multiple_of(base + g * GK, GK)
            pltpu.async_copy(bufs[b], g_hbm.at[pl.ds(off, GK)], wsems[b])

        def wait_wb(b):
            pltpu.make_async_copy(
                bufs[b], g_hbm.at[pl.ds(0, GK)], wsems[b]).wait()

        for b in range(NB):
            start_gather(b, b)

        def body(r, carry):
            for b in range(NB):
                g = r * NB + b
                wait_gather(b, g)
                start_wb(b, g)
            for b in range(NB):
                wait_wb(b)
                start_gather(b, (r + 1) * NB + b)
            return carry

        lax.fori_loop(0, rounds - 1, body, 0)
        for b in range(NB):
            g = (rounds - 1) * NB + b
            wait_gather(b, g)
            start_wb(b, g)
        for b in range(NB):
            wait_wb(b)

    return kfn


def _unpack_words(w):
    wu = w.astype(jnp.uint32)
    lo = (wu & jnp.uint32(0xFFFF)).astype(jnp.uint16)
    hi = (wu >> jnp.uint32(16)).astype(jnp.uint16)
    return (lax.bitcast_convert_type(lo, jnp.bfloat16),
            lax.bitcast_convert_type(hi, jnp.bfloat16))


def _pack_words(yl, yh):
    lo = lax.bitcast_convert_type(
        yl.astype(jnp.bfloat16), jnp.uint16).astype(jnp.uint32)
    hi = lax.bitcast_convert_type(
        yh.astype(jnp.bfloat16), jnp.uint16).astype(jnp.uint32)
    return lax.bitcast_convert_type(lo | (hi << jnp.uint32(16)), jnp.int32)


def _mm(xp, Ws, bias8=None, relu=False, res=None, mask=True, row_off=0,
        unpack_in=False, out=("val",), out_dtype=jnp.bfloat16, BM=256):
    M, K = xp.shape
    N = Ws[0].shape[1]
    nblk = M // BM
    args = [xp] + list(Ws)
    in_specs = [pl.BlockSpec((BM, K), lambda i: (i, 0))] + [
        pl.BlockSpec(W.shape, lambda i: (0, 0)) for W in Ws]
    if bias8 is not None:
        args.append(bias8)
        in_specs.append(pl.BlockSpec((8, N), lambda i: (0, 0)))
    if res is not None:
        args.append(res)
        in_specs.append(pl.BlockSpec((BM, N), lambda i: (i, 0)))
    have_bias = bias8 is not None
    have_res = res is not None
    nw = len(Ws)

    def body(*refs):
        x_ref = refs[0]
        w_refs = refs[1:1 + nw]
        rest = refs[1 + nw:-len(out)]
        o_refs = refs[-len(out):]
        if unpack_in:
            lo, hi = _unpack_words(x_ref[...])
            y = jnp.dot(lo, w_refs[0][...], preferred_element_type=jnp.float32)
            y = y + jnp.dot(hi, w_refs[1][...],
                            preferred_element_type=jnp.float32)
        else:
            y = jnp.dot(x_ref[...].astype(jnp.bfloat16), w_refs[0][...],
                        preferred_element_type=jnp.float32)
        ri = 0
        if have_bias:
            y = y + rest[0][0:1, :]
            ri = 1
        if mask:
            i = pl.program_id(0)
            rows = row_off + i * BM + lax.broadcasted_iota(
                jnp.int32, (BM, 1), 0)
            y = jnp.where(rows < V, y, 0.0)
        if have_res:
            y = y + rest[ri][...].astype(jnp.float32)
        if relu:
            y = jnp.maximum(y, 0.0)
        for kind, o_ref in zip(out, o_refs):
            if kind == "pack":
                o_ref[...] = _pack_words(y[:, :N // 2], y[:, N // 2:])
            else:
                o_ref[...] = y.astype(o_ref.dtype)

    out_specs = []
    out_shapes = []
    for kind in out:
        if kind == "pack":
            out_specs.append(pl.BlockSpec((BM, N // 2), lambda i: (i, 0)))
            out_shapes.append(jax.ShapeDtypeStruct((M, N // 2), jnp.int32))
        else:
            out_specs.append(pl.BlockSpec((BM, N), lambda i: (i, 0)))
            out_shapes.append(jax.ShapeDtypeStruct((M, N), out_dtype))
    r = pl.pallas_call(
        body,
        grid=(nblk,),
        in_specs=in_specs,
        out_specs=out_specs if len(out) > 1 else out_specs[0],
        out_shape=out_shapes if len(out) > 1 else out_shapes[0],
    )(*args)
    return r


def _wd_and_max(fs, Wd, BM=256):
    M, K = fs.shape
    N = Wd.shape[1]
    nblk = M // BM

    def body(x_ref, w_ref, o_ref, m_ref):
        i = pl.program_id(0)
        y = jnp.dot(x_ref[...], w_ref[...], preferred_element_type=jnp.float32)
        rows = i * BM + lax.broadcasted_iota(jnp.int32, (BM, 1), 0)
        valid = rows < V
        yv = jnp.where(valid, y, 0.0)
        o_ref[...] = yv
        ym = jnp.where(valid, y, -1e30)
        m = ym[0:8]
        for j in range(1, BM // 8):
            m = jnp.maximum(m, ym[j * 8:(j + 1) * 8])

        @pl.when(i == 0)
        def _():
            m_ref[...] = m

        @pl.when(i > 0)
        def _():
            m_ref[...] = jnp.maximum(m_ref[...], m)

    return pl.pallas_call(
        body,
        grid=(nblk,),
        in_specs=[
            pl.BlockSpec((BM, K), lambda i: (i, 0)),
            pl.BlockSpec((K, N), lambda i: (0, 0)),
        ],
        out_specs=[
            pl.BlockSpec((BM, N), lambda i: (i, 0)),
            pl.BlockSpec((8, N), lambda i: (0, 0)),
        ],
        out_shape=[
            jax.ShapeDtypeStruct((M, N), jnp.float32),
            jax.ShapeDtypeStruct((8, N), jnp.float32),
        ],
    )(fs, Wd)


def _final_mlp(pfs, fs, macc, W1p, W1f, W1g, b1_8, W2, b2_8, W3p, b3_8,
               BM=256):
    M = pfs.shape[0]
    nblk = M // BM

    def body(p_ref, f_ref, g_ref, w1p, w1f, w1g, b1, w2, b2, w3, b3, o_ref):
        gmax = jnp.max(g_ref[...], axis=0, keepdims=True)
        gc = jnp.dot(gmax.astype(jnp.bfloat16), w1g[...],
                     preferred_element_type=jnp.float32)
        o1 = jnp.dot(p_ref[...], w1p[...], preferred_element_type=jnp.float32)
        o1 = o1 + jnp.dot(f_ref[...].astype(jnp.bfloat16), w1f[...],
                          preferred_element_type=jnp.float32)
        o1 = jnp.maximum(o1 + gc + b1[0:1, :], 0.0).astype(jnp.bfloat16)
        o2 = jnp.maximum(
            jnp.dot(o1, w2[...], preferred_element_type=jnp.float32)
            + b2[0:1, :], 0.0).astype(jnp.bfloat16)
        o_ref[...] = (
            jnp.dot(o2, w3[...], preferred_element_type=jnp.float32)
            + b3[0:1, :])

    specs = [
        pl.BlockSpec((BM, 256), lambda i: (i, 0)),
        pl.BlockSpec((BM, 128), lambda i: (i, 0)),
        pl.BlockSpec((8, 128), lambda i: (0, 0)),
        pl.BlockSpec((256, 256), lambda i: (0, 0)),
        pl.BlockSpec((128, 256), lambda i: (0, 0)),
        pl.BlockSpec((128, 256), lambda i: (0, 0)),
        pl.BlockSpec((8, 256), lambda i: (0, 0)),
        pl.BlockSpec((256, 128), lambda i: (0, 0)),
        pl.BlockSpec((8, 128), lambda i: (0, 0)),
        pl.BlockSpec((128, 128), lambda i: (0, 0)),
        pl.BlockSpec((8, 128), lambda i: (0, 0)),
    ]
    return pl.pallas_call(
        body,
        grid=(nblk,),
        in_specs=specs,
        out_specs=pl.BlockSpec((BM, 128), lambda i: (i, 0)),
        out_shape=jax.ShapeDtypeStruct((M, 128), jnp.float32),
    )(pfs, fs, macc, W1p, W1f, W1g, b1_8, W2, b2_8, W3p, b3_8)


def _b8(b):
    return jnp.broadcast_to(b.reshape(1, -1), (8, b.shape[0]))


def _bf(a):
    return a.astype(jnp.bfloat16)


def kernel(x, spiral, Wp, W1a, b1a, W1b, b1b, Wd, Wr0a, br0a, Wr0b, br0b,
           Wr1a, br1a, Wr1b, br1b, Wr2a, br2a, Wr2b, br2b, Wo1, bo1, Wo2,
           bo2, Wo3, bo3):
    Bn, Vn, FIN = x.shape
    KP = 512
    xp = _bf(jnp.pad(x[0], ((0, VP - Vn), (0, KP - FIN))))
    Wpp = _bf(jnp.pad(Wp, ((0, KP - FIN), (0, 0))))
    idxf = jnp.pad(spiral.reshape(-1), (0, VP * L - spiral.size))
    idxf = idxf.astype(jnp.int32)

    NCH = 4
    VPc = VP // NCH
    Rc = VP * L // NCH
    gather512b = _make_sc_gather(128, NCH)
    idxc = [idxf[c * Rc:(c + 1) * Rc] for c in range(NCH)]

    def spiral_conv(tbl_words, Ws, bias8, unpack, res=None, relu=True,
                    out=("val",), out_dtype=jnp.bfloat16):
        gs = [gather512b(tbl_words, idxc[c]) for c in range(NCH)]
        K = gs[0].shape[1] * L
        outs = []
        for c in range(NCH):
            rsl = None if res is None else res[c * VPc:(c + 1) * VPc]
            xin = gs[c] if unpack else lax.bitcast_convert_type(
                gs[c], jnp.float32)
            outs.append(_mm(xin.reshape(VPc, K), Ws, bias8=bias8,
                            relu=relu, res=rsl, row_off=c * VPc,
                            unpack_in=unpack, out=out, out_dtype=out_dtype))
        if len(out) > 1:
            return [jnp.concatenate([o[j] for o in outs])
                    for j in range(len(out))]
        return jnp.concatenate(outs)

    def lohi(W):
        Wr = W.reshape(L, 256, W.shape[1])
        return [_bf(Wr[:, :128].reshape(L * 128, -1)),
                _bf(Wr[:, 128:].reshape(L * 128, -1))]

    pfs_w, pfs = _mm(xp, [Wpp], relu=True, out=("pack", "val"))

    h_w = spiral_conv(pfs_w, lohi(W1a), _b8(b1a), True, out=("pack",))
    fs = spiral_conv(h_w, lohi(W1b), _b8(b1b), True, res=pfs)

    fs, macc = _wd_and_max(fs, _bf(Wd))

    for (Wa, ba, Wb, bb) in ((Wr0a, br0a, Wr0b, br0b),
                             (Wr1a, br1a, Wr1b, br1b),
                             (Wr2a, br2a, Wr2b, br2b)):
        h = spiral_conv(lax.bitcast_convert_type(fs, jnp.int32),
                        [_bf(Wa)], _b8(ba), False, out_dtype=jnp.float32)
        fs = spiral_conv(lax.bitcast_convert_type(h, jnp.int32),
                         [_bf(Wb)], _b8(bb), False, res=fs,
                         out_dtype=jnp.float32)

    W1p = _bf(Wo1[:256])
    W1f = _bf(Wo1[256:384])
    W1g = _bf(Wo1[384:])
    W3p = _bf(jnp.pad(Wo3, ((0, 0), (0, 128 - Wo3.shape[1]))))
    b3p = jnp.pad(bo3, (0, 128 - bo3.shape[0]))
    o = _final_mlp(pfs, fs, macc, W1p, W1f, W1g, _b8(bo1), _bf(Wo2),
                   _b8(bo2), W3p, _b8(b3p))
    return o[:V, :3].reshape(1, V, 3)
